# baseline (speedup 1.0000x reference)
import jax
import jax.numpy as jnp
from jax.experimental import pallas as pl
from jax.experimental.pallas import tpu as pltpu

_BLK = 256
_NSEM = 8


def _body(pe_ref, o_ref, scratch_ref, sems):
    scratch_ref[...] = jnp.broadcast_to(pe_ref[...], scratch_ref.shape)
    n = o_ref.shape[0] // _BLK
    for i in range(n):
        pltpu.make_async_copy(
            scratch_ref,
            o_ref.at[pl.ds(i * _BLK, _BLK), :],
            sems.at[i % _NSEM],
        ).start()
    for i in range(n):
        pltpu.make_async_copy(
            scratch_ref,
            o_ref.at[pl.ds(i * _BLK, _BLK), :],
            sems.at[i % _NSEM],
        ).wait()


def kernel(x, pos_emb):
    B, T = x.shape
    D = pos_emb.shape[1]
    pe = pos_emb[:T].reshape(1, T * D)
    out = pl.pallas_call(
        _body,
        in_specs=[pl.BlockSpec((1, T * D), lambda: (0, 0))],
        out_specs=pl.BlockSpec(memory_space=pl.ANY),
        out_shape=jax.ShapeDtypeStruct((B, T * D), pos_emb.dtype),
        scratch_shapes=[
            pltpu.VMEM((_BLK, T * D), pos_emb.dtype),
            pltpu.SemaphoreType.DMA((_NSEM,)),
        ],
    )(pe)
    return out.reshape(B, T, D)
